# baseline (speedup 1.0000x reference)
import functools

import jax
import jax.numpy as jnp
from jax import lax
from jax.experimental import pallas as pl
from jax.experimental.pallas import tpu as pltpu
from jax.experimental.pallas import tpu_sc as plsc

_NC = 2
_NS = 16
_NW = _NC * _NS
_L = 16

_B = 4
_R = 8192
_C = 4096
_N = _B * _C
_PER_W = _N // _NW
_CHUNKS = 8
_CHUNK = _PER_W // _CHUNKS


def _gather_body(x_hbm, idx_hbm, out_hbm, idx_v, fidx_v, out_v,
                 idx_sem, sem, out_sem):
    wid = lax.axis_index("s") * _NC + lax.axis_index("c")
    e0 = wid * _PER_W
    b = e0 // _C
    cbase = e0 % _C

    lane = lax.iota(jnp.int32, _L)

    idx_copies = [
        pltpu.async_copy(
            idx_hbm.at[b, 0, pl.ds(cbase + j * _CHUNK, _CHUNK)],
            idx_v.at[pl.ds(j * _CHUNK, _CHUNK)],
            idx_sem,
        )
        for j in range(_CHUNKS)
    ]

    gathers = []
    for j in range(_CHUNKS):
        idx_copies[j].wait()
        for i in range(_CHUNK // _L):
            c = cbase + j * _CHUNK + i * _L
            v = idx_v[pl.ds(j * _CHUNK + i * _L, _L)]
            r = v + b * _R
            addr = (
                (r >> 3) * 32768
                + (r & 7) * 128
                + ((c >> 7) * 1024 + (c & 127))
                + lane
            )
            fidx_v[pl.ds(j * _CHUNK + i * _L, _L)] = addr
        gathers.append(
            pltpu.async_copy(
                x_hbm.at[fidx_v.at[pl.ds(j * _CHUNK, _CHUNK)]],
                out_v.at[pl.ds(j * _CHUNK, _CHUNK)],
                sem,
            )
        )

    out_copies = []
    for j in range(_CHUNKS):
        gathers[j].wait()
        out_copies.append(
            pltpu.async_copy(
                out_v.at[pl.ds(j * _CHUNK, _CHUNK)],
                out_hbm.at[b, 0, pl.ds(cbase + j * _CHUNK, _CHUNK)],
                out_sem,
            )
        )
    for cp in out_copies:
        cp.wait()


@jax.jit
def kernel(x, index):
    x_phys = (
        x.reshape(_B * _R // 8, 8, _C // 128, 128)
        .transpose(0, 2, 1, 3)
        .reshape(-1)
    )

    mesh = plsc.VectorSubcoreMesh(core_axis_name="c", subcore_axis_name="s")
    run = functools.partial(
        pl.kernel,
        mesh=mesh,
        out_type=jax.ShapeDtypeStruct((_B, 1, _C), jnp.float32),
        scratch_types=[
            pltpu.VMEM((_PER_W,), jnp.int32),
            pltpu.VMEM((_PER_W,), jnp.int32),
            pltpu.VMEM((_PER_W,), jnp.float32),
            pltpu.SemaphoreType.DMA,
            pltpu.SemaphoreType.DMA,
            pltpu.SemaphoreType.DMA,
        ],
        compiler_params=pltpu.CompilerParams(
            disable_bounds_checks=True,
            disable_semaphore_checks=True,
            skip_device_barrier=True,
        ),
    )(_gather_body)
    return run(x_phys, index)


# speedup vs baseline: 1.0094x; 1.0094x over previous
import functools

import jax
import jax.numpy as jnp
from jax import lax
from jax.experimental import pallas as pl
from jax.experimental.pallas import tpu as pltpu
from jax.experimental.pallas import tpu_sc as plsc

_NC = 2
_NS = 16
_NW = _NC * _NS
_L = 16

_B = 4
_R = 8192
_C = 4096
_N = _B * _C
_PER_W = _N // _NW
_CHUNKS = 4
_CHUNK = _PER_W // _CHUNKS


def _gather_body(x_hbm, idx_hbm, out_hbm, idx_v, fidx_v, out_v,
                 idx_sem, sem, out_sem):
    wid = lax.axis_index("s") * _NC + lax.axis_index("c")
    e0 = wid * _PER_W
    b = e0 // _C
    cbase = e0 % _C

    lane = lax.iota(jnp.int32, _L)

    idx_copies = [
        pltpu.async_copy(
            idx_hbm.at[b, 0, pl.ds(cbase + j * _CHUNK, _CHUNK)],
            idx_v.at[pl.ds(j * _CHUNK, _CHUNK)],
            idx_sem,
        )
        for j in range(_CHUNKS)
    ]

    gathers = []
    for j in range(_CHUNKS):
        idx_copies[j].wait()
        for i in range(_CHUNK // _L):
            c = cbase + j * _CHUNK + i * _L
            v = idx_v[pl.ds(j * _CHUNK + i * _L, _L)]
            r = v + b * _R
            addr = (
                (r >> 3) * 32768
                + (r & 7) * 128
                + ((c >> 7) * 1024 + (c & 127))
                + lane
            )
            fidx_v[pl.ds(j * _CHUNK + i * _L, _L)] = addr
        gathers.append(
            pltpu.async_copy(
                x_hbm.at[fidx_v.at[pl.ds(j * _CHUNK, _CHUNK)]],
                out_v.at[pl.ds(j * _CHUNK, _CHUNK)],
                sem,
            )
        )

    out_copies = []
    for j in range(_CHUNKS):
        gathers[j].wait()
        out_copies.append(
            pltpu.async_copy(
                out_v.at[pl.ds(j * _CHUNK, _CHUNK)],
                out_hbm.at[b, 0, pl.ds(cbase + j * _CHUNK, _CHUNK)],
                out_sem,
            )
        )
    for cp in out_copies:
        cp.wait()


@jax.jit
def kernel(x, index):
    x_phys = (
        x.reshape(_B * _R // 8, 8, _C // 128, 128)
        .transpose(0, 2, 1, 3)
        .reshape(-1)
    )

    mesh = plsc.VectorSubcoreMesh(core_axis_name="c", subcore_axis_name="s")
    run = functools.partial(
        pl.kernel,
        mesh=mesh,
        out_type=jax.ShapeDtypeStruct((_B, 1, _C), jnp.float32),
        scratch_types=[
            pltpu.VMEM((_PER_W,), jnp.int32),
            pltpu.VMEM((_PER_W,), jnp.int32),
            pltpu.VMEM((_PER_W,), jnp.float32),
            pltpu.SemaphoreType.DMA,
            pltpu.SemaphoreType.DMA,
            pltpu.SemaphoreType.DMA,
        ],
        compiler_params=pltpu.CompilerParams(
            disable_bounds_checks=True,
            disable_semaphore_checks=True,
            skip_device_barrier=True,
        ),
    )(_gather_body)
    return run(x_phys, index)
